# baseline (speedup 1.0000x reference)
import jax
import jax.numpy as jnp
from jax.experimental import pallas as pl
from jax.experimental.pallas import tpu as pltpu

_B, _N, _D = 8, 1024, 256
_T = 128
_DN_T = (((1,), (1,)), ((), ()))


def _loss_body(feat_ref, pts_ref, pptsT_ref, tri_ref, out_ref, fn_scr):
    i = pl.program_id(0)
    j = pl.program_id(1)

    @pl.when((i == 0) & (j == 0))
    def _():
        out_ref[0, 0] = 0.0
        for b in range(_B):
            f = feat_ref[b]
            n2 = jnp.sum(f * f, axis=-1, keepdims=True)
            norm = jnp.maximum(jnp.sqrt(n2), 1e-8)
            fn_scr[b] = (f / norm).astype(jnp.bfloat16)

    acc = jnp.float32(0.0)
    for tn in range(_N // _T):
        a = pts_ref[i, pl.ds(tn * _T, _T)]
        a2 = jnp.sum(a * a, axis=-1, keepdims=True)
        fj = fn_scr[j, pl.ds(tn * _T, _T)]
        for tm in range(tn, _N // _T):
            pT = pptsT_ref[i, j, :, pl.ds(tm * _T, _T)]
            b2 = jnp.sum(pT * pT, axis=0, keepdims=True)
            ab = jax.lax.dot(a, pT, preferred_element_type=jnp.float32)
            d2 = a2 + b2 - 2.0 * ab
            mask = d2 <= 1.0
            if tn == tm:
                mask = mask & tri_ref[...]
            fi = fn_scr[i, pl.ds(tm * _T, _T)]
            c = jax.lax.dot_general(fj, fi, _DN_T,
                                    preferred_element_type=jnp.float32)
            acc += jnp.sum(jnp.where(mask, 1.0 - c, 0.0))

    out_ref[0, 0] += acc


def _match_loss(features, pts, pptsT, tri):
    out = pl.pallas_call(
        _loss_body,
        grid=(_B, _B),
        in_specs=[
            pl.BlockSpec((_B, _N, _D), lambda i, j: (0, 0, 0)),
            pl.BlockSpec((_B, _N, 2), lambda i, j: (0, 0, 0)),
            pl.BlockSpec((_B, _B, 2, _N), lambda i, j: (0, 0, 0, 0)),
            pl.BlockSpec((_T, _T), lambda i, j: (0, 0)),
        ],
        out_specs=pl.BlockSpec(memory_space=pltpu.SMEM),
        out_shape=jax.ShapeDtypeStruct((1, 1), jnp.float32),
        scratch_shapes=[pltpu.VMEM((_B, _N, _D), jnp.bfloat16)],
    )(features, pts, pptsT, tri)
    return out[0, 0]


def kernel(features, points, proj_pts, invis_idx, height, width):
    factor = jnp.array([(width - 1.0) / 2.0, (height - 1.0) / 2.0],
                       dtype=points.dtype)
    pts = (points + 1.0) * factor
    ppts = (proj_pts + 1.0) * factor
    pptsT = jnp.swapaxes(ppts, 2, 3)

    tri = jnp.triu(jnp.ones((_T, _T), dtype=jnp.bool_))
    return _match_loss(features, pts, pptsT, tri)


# speedup vs baseline: 1.0253x; 1.0253x over previous
import jax
import jax.numpy as jnp
from jax.experimental import pallas as pl
from jax.experimental.pallas import tpu as pltpu

_B, _N, _D = 8, 1024, 256
_T = 256
_DN_T = (((1,), (1,)), ((), ()))


def _loss_body(feat_ref, pts_ref, pptsT_ref, tri_ref, out_ref, fn_scr):
    i = pl.program_id(0)
    j = pl.program_id(1)

    @pl.when((i == 0) & (j == 0))
    def _():
        out_ref[0, 0] = 0.0
        for b in range(_B):
            f = feat_ref[b]
            n2 = jnp.sum(f * f, axis=-1, keepdims=True)
            norm = jnp.maximum(jnp.sqrt(n2), 1e-8)
            fn_scr[b] = (f / norm).astype(jnp.bfloat16)

    acc = jnp.float32(0.0)
    for tn in range(_N // _T):
        w = _N - tn * _T
        a = pts_ref[i, pl.ds(tn * _T, _T)]
        a2 = jnp.sum(a * a, axis=-1, keepdims=True)
        fj = fn_scr[j, pl.ds(tn * _T, _T)]
        pT = pptsT_ref[i, j, :, pl.ds(tn * _T, w)]
        b2 = jnp.sum(pT * pT, axis=0, keepdims=True)
        ab = jax.lax.dot(a, pT, preferred_element_type=jnp.float32)
        d2 = a2 + b2 - 2.0 * ab
        mask = (d2 <= 1.0) & tri_ref[:, pl.ds(0, w)]
        fi = fn_scr[i, pl.ds(tn * _T, w)]
        c = jax.lax.dot_general(fj, fi, _DN_T,
                                preferred_element_type=jnp.float32)
        acc += jnp.sum(jnp.where(mask, 1.0 - c, 0.0))

    out_ref[0, 0] += acc


def _match_loss(features, pts, pptsT, tri):
    out = pl.pallas_call(
        _loss_body,
        grid=(_B, _B),
        in_specs=[
            pl.BlockSpec((_B, _N, _D), lambda i, j: (0, 0, 0)),
            pl.BlockSpec((_B, _N, 2), lambda i, j: (0, 0, 0)),
            pl.BlockSpec((_B, _B, 2, _N), lambda i, j: (0, 0, 0, 0)),
            pl.BlockSpec((_T, _N), lambda i, j: (0, 0)),
        ],
        out_specs=pl.BlockSpec(memory_space=pltpu.SMEM),
        out_shape=jax.ShapeDtypeStruct((1, 1), jnp.float32),
        scratch_shapes=[pltpu.VMEM((_B, _N, _D), jnp.bfloat16)],
    )(features, pts, pptsT, tri)
    return out[0, 0]


def kernel(features, points, proj_pts, invis_idx, height, width):
    factor = jnp.array([(width - 1.0) / 2.0, (height - 1.0) / 2.0],
                       dtype=points.dtype)
    pts = (points + 1.0) * factor
    ppts = (proj_pts + 1.0) * factor
    pptsT = jnp.swapaxes(ppts, 2, 3)

    tri = jnp.concatenate(
        [jnp.triu(jnp.ones((_T, _T), dtype=jnp.bool_)),
         jnp.ones((_T, _N - _T), dtype=jnp.bool_)], axis=1)
    return _match_loss(features, pts, pptsT, tri)


# speedup vs baseline: 1.2257x; 1.1955x over previous
import jax
import jax.numpy as jnp
from jax.experimental import pallas as pl
from jax.experimental.pallas import tpu as pltpu

_B, _N, _D = 8, 1024, 256
_T = 256
_DN_T = (((1,), (1,)), ((), ()))


def _loss_body(feat_ref, pts_ref, pptsT_ref, tri_ref, out_ref, fn_scr):
    i = pl.program_id(0)
    j = pl.program_id(1)

    @pl.when((i == 0) & (j == 0))
    def _():
        out_ref[0, 0] = 0.0
        for b in range(_B):
            f = feat_ref[b]
            n2 = jnp.sum(f * f, axis=-1, keepdims=True)
            norm = jnp.maximum(jnp.sqrt(n2), 1e-8)
            fn_scr[b] = (f / norm).astype(jnp.bfloat16)

    acc = jnp.float32(0.0)
    for tn in range(_N // _T):
        a = pts_ref[i, pl.ds(tn * _T, _T)]
        a2 = jnp.sum(a * a, axis=-1, keepdims=True)
        fj = fn_scr[j, pl.ds(tn * _T, _T)]
        for tm in range(tn, _N // _T):
            pT = pptsT_ref[i, j, :, pl.ds(tm * _T, _T)]
            b2 = jnp.sum(pT * pT, axis=0, keepdims=True)
            ab = jax.lax.dot(a, pT, preferred_element_type=jnp.float32)
            d2 = a2 + b2 - 2.0 * ab
            mask = d2 <= 1.0
            if tn == tm:
                mask = mask & tri_ref[...]
            fi = fn_scr[i, pl.ds(tm * _T, _T)]
            c = jax.lax.dot_general(fj, fi, _DN_T,
                                    preferred_element_type=jnp.float32)
            acc += jnp.sum(jnp.where(mask, 1.0 - c, 0.0))

    out_ref[0, 0] += acc


def _match_loss(features, pts, pptsT, tri):
    out = pl.pallas_call(
        _loss_body,
        grid=(_B, _B),
        in_specs=[
            pl.BlockSpec((_B, _N, _D), lambda i, j: (0, 0, 0)),
            pl.BlockSpec((_B, _N, 2), lambda i, j: (0, 0, 0)),
            pl.BlockSpec((_B, _B, 2, _N), lambda i, j: (0, 0, 0, 0)),
            pl.BlockSpec((_T, _T), lambda i, j: (0, 0)),
        ],
        out_specs=pl.BlockSpec(memory_space=pltpu.SMEM),
        out_shape=jax.ShapeDtypeStruct((1, 1), jnp.float32),
        scratch_shapes=[pltpu.VMEM((_B, _N, _D), jnp.bfloat16)],
    )(features, pts, pptsT, tri)
    return out[0, 0]


def kernel(features, points, proj_pts, invis_idx, height, width):
    factor = jnp.array([(width - 1.0) / 2.0, (height - 1.0) / 2.0],
                       dtype=points.dtype)
    pts = (points + 1.0) * factor
    ppts = (proj_pts + 1.0) * factor
    pptsT = jnp.swapaxes(ppts, 2, 3)

    tri = jnp.triu(jnp.ones((_T, _T), dtype=jnp.bool_))
    return _match_loss(features, pts, pptsT, tri)
